# baseline (speedup 1.0000x reference)
import functools

import jax
import jax.numpy as jnp
from jax import lax
from jax.experimental import pallas as pl
from jax.experimental.pallas import tpu as pltpu
from jax.experimental.pallas import tpu_sc as plsc

_N_FIELDS = 26
_BATCH = 4096
_VOCAB = 100000
_DIM = 64
_B = _N_FIELDS * _BATCH
_NW = 32
_NCOLS = _N_FIELDS * _VOCAB

_WIN = 256
_NWIN = (_NCOLS + _WIN - 1) // _WIN
_LAST_G = _NWIN - 1
_WPW = 318

_ACAP = 3840
_NBUCK = 20
_BCAP = 368
_SENT = 0x7FFFFFF0

_IDXC = 8
_IDXR = 104

_M_ROWS = _NW * _ACAP
_PERM_N = _B + 256

_CPW = _B // (_NW * 128)


def _iota16():
    return lax.broadcasted_iota(jnp.int32, (16,), 0)


def _popcount(mask):
    return plsc.all_reduce_population_count(mask)[0]


def _cstore(ref_a, xa, ref_b, xb, m, off, trash):
    rank = plsc.cumsum(m.astype(jnp.int32))
    pos = jnp.where(m, off + rank - 1, trash)
    plsc.store_scatter(ref_a, [pos], xa)
    if ref_b is not None:
        plsc.store_scatter(ref_b, [pos], xb)
    return rank[15]


def _build_k1():
    mesh = plsc.VectorSubcoreMesh(core_axis_name="c", subcore_axis_name="s")

    @functools.partial(
        pl.kernel,
        mesh=mesh,
        compiler_params=pltpu.CompilerParams(
            use_tc_tiling_on_sc=True, needs_layout_passes=False),
        out_type=(
            jax.ShapeDtypeStruct((_M_ROWS, 128), jnp.float32),
            jax.ShapeDtypeStruct((_NW, _ACAP // 128, 128), jnp.int32),
        ),
        scratch_types=[
            pltpu.VMEM((_IDXR * 128,), jnp.int32),
            pltpu.VMEM((_IDXR * 128,), jnp.int32),
            pltpu.VMEM((_ACAP,), jnp.int32),
            pltpu.VMEM((_ACAP,), jnp.int32),
            pltpu.VMEM((_NBUCK * _BCAP,), jnp.int32),
            pltpu.VMEM((_NBUCK * _BCAP,), jnp.int32),
            pltpu.VMEM((64, _WIN), jnp.float32),
            pltpu.VMEM((64, _WIN), jnp.float32),
            pltpu.VMEM((256, 128), jnp.float32),
            pltpu.VMEM((64,), jnp.int32),
            pltpu.VMEM((_ACAP,), jnp.int32),
            pltpu.VMEM((_ACAP // 128, 128), jnp.int32),
            pltpu.SemaphoreType.DMA,
            pltpu.SemaphoreType.DMA,
        ],
    )
    def k1(idx_hbm, tp_hbm, tail_hbm, inter_hbm, jout_hbm,
           idxbuf0, idxbuf1, rstage, jstage, rbuck, jbuck, slab0, slab1,
           outstage, winr, j1d, jrep, isem, ssem):
        wid = lax.axis_index("s") * 2 + lax.axis_index("c")
        lo = wid * (_WPW * _WIN)
        hi = jnp.minimum(lo + _WPW * _WIN, _NCOLS)
        dumpj = _B + wid
        iota = _iota16()
        idxbufs = (idxbuf0, idxbuf1)
        slabs = (slab0, slab1)

        for v in range(_ACAP // 16):
            rstage[pl.ds(v * 16, 16)] = jnp.full((16,), _SENT, jnp.int32)
            j1d[pl.ds(v * 16, 16)] = jnp.full((16,), dumpj, jnp.int32)
        for v in range(_NBUCK * _BCAP // 16):
            rbuck[pl.ds(v * 16, 16)] = jnp.full((16,), _SENT, jnp.int32)
        for v in range(4):
            winr[pl.ds(v * 16, 16)] = jnp.zeros((16,), jnp.int32)

        pltpu.async_copy(idx_hbm.at[pl.ds(0, _IDXR * 128)], idxbuf0, isem)

        def chunk_body(c, cnt):
            for pb in range(2):
                cc = c * 2 + pb
                src = idx_hbm.at[pl.ds(
                    pl.multiple_of(cc * _IDXR * 128, 1024), _IDXR * 128)]
                pltpu.make_async_copy(src, idxbufs[pb], isem).wait()

                @pl.when(cc + 1 < _IDXC)
                def _():
                    nsrc = idx_hbm.at[pl.ds(
                        pl.multiple_of((cc + 1) * _IDXR * 128, 1024),
                        _IDXR * 128)]
                    pltpu.async_copy(nsrc, idxbufs[1 - pb], isem)

                def row_body(rr, cnt2):
                    grow = cc * _IDXR + rr
                    off = (grow // 32) * _VOCAB
                    for sv in range(8):
                        r = idxbufs[pb][pl.ds(
                            pl.multiple_of(rr * 128 + sv * 16, 16), 16)] + off
                        m = (r >= lo) & (r < hi)
                        jv = grow * 128 + sv * 16 + iota
                        o = jnp.minimum(cnt2, _ACAP - 32)
                        nc = _cstore(rstage, r, jstage, jv, m, o, _ACAP - 1)
                        cnt2 = cnt2 + nc
                    return cnt2

                cnt = lax.fori_loop(0, _IDXR, row_body, cnt)
            return cnt

        cnt = lax.fori_loop(0, _IDXC // 2, chunk_body, jnp.int32(0))
        rstage[pl.ds(_ACAP - 16, 16)] = jnp.full((16,), _SENT, jnp.int32)

        def buck_body(v, cnts):
            p16 = pl.multiple_of(v * 16, 16)
            r = rstage[pl.ds(p16, 16)]
            jv = jstage[pl.ds(p16, 16)]
            bid = (r - lo) >> 12
            new = []
            for b in range(_NBUCK):
                m = bid == b
                o = b * _BCAP + jnp.minimum(cnts[b], _BCAP - 32)
                nc = _cstore(rbuck, r, jbuck, jv, m, o, b * _BCAP + _BCAP - 1)
                new.append(cnts[b] + nc)
            return tuple(new)

        lax.fori_loop(0, _ACAP // 16, buck_body,
                      tuple(jnp.int32(0) for _ in range(_NBUCK)))
        for b in range(_NBUCK):
            rbuck[pl.ds(b * _BCAP + _BCAP - 16, 16)] = jnp.full(
                (16,), _SENT, jnp.int32)

        def issue_slab(k_w, pb):
            g = wid * _WPW + k_w

            @pl.when(g < _LAST_G)
            def _():
                pltpu.async_copy(
                    tp_hbm.at[:, pl.ds(pl.multiple_of(g * _WIN, _WIN), _WIN)],
                    slabs[pb], ssem)

            @pl.when(g == _LAST_G)
            def _():
                pltpu.async_copy(
                    tail_hbm, slabs[pb].at[:, pl.ds(0, 128)], ssem)

        def wait_slab(k_w, pb):
            g = wid * _WPW + k_w

            @pl.when(g < _LAST_G)
            def _():
                pltpu.make_async_copy(
                    tp_hbm.at[:, pl.ds(pl.multiple_of(g * _WIN, _WIN), _WIN)],
                    slabs[pb], ssem).wait()

            @pl.when(g == _LAST_G)
            def _():
                pltpu.make_async_copy(
                    tail_hbm, slabs[pb].at[:, pl.ds(0, 128)], ssem).wait()

        @pl.when(wid * _WPW < _NWIN)
        def _():
            issue_slab(0, 0)

        def win_body(k2, base):
            for pb in range(2):
                k_w = k2 * 2 + pb
                g = wid * _WPW + k_w
                wait_slab(k_w, pb)

                @pl.when(k_w + 1 < _WPW)
                def _():
                    @pl.when(g + 1 < _NWIN)
                    def _():
                        issue_slab(k_w + 1, 1 - pb)

                bk = k_w >> 4

                def scan_body(v, wcnt):
                    p = pl.multiple_of(bk * _BCAP + v * 16, 16)
                    r = rbuck[pl.ds(p, 16)]
                    jv = jbuck[pl.ds(p, 16)]
                    m = (r >> 8) == g
                    rcol = r & 255
                    o = jnp.minimum(wcnt, 47)
                    nc = _cstore(winr, rcol, None, None, m, o, 63)
                    ob = jnp.minimum(base + wcnt, _ACAP - 32)
                    _cstore(j1d, jv, None, None, m, ob, _ACAP - 1)
                    return wcnt + nc

                wcnt = lax.fori_loop(0, _BCAP // 16, scan_body, jnp.int32(0))

                def ext_body(gi, _):
                    rcol = winr[pl.ds(pl.multiple_of(gi * 16, 16), 16)]
                    rows = (base + gi * 16 + iota) & 255
                    for d in range(64):
                        dvec = jnp.full((16,), d, jnp.int32)
                        vals = plsc.load_gather(slabs[pb], [dvec, rcol])
                        plsc.store_scatter(outstage, [rows, dvec], vals)
                    return _

                lax.fori_loop(0, (wcnt + 15) >> 4, ext_body, 0)

                nbase = base + wcnt

                @pl.when((nbase >> 7) > (base >> 7))
                def _():
                    blk = base >> 7
                    pltpu.sync_copy(
                        outstage.at[pl.ds(
                            pl.multiple_of((blk & 1) * 128, 128), 128)],
                        inter_hbm.at[pl.ds(
                            pl.multiple_of(wid * _ACAP + blk * 128, 128),
                            128)])

                base = nbase
            return base

        base = lax.fori_loop(0, _WPW // 2, win_body, jnp.int32(0))

        @pl.when((base & 127) != 0)
        def _():
            blk = base >> 7
            pltpu.sync_copy(
                outstage.at[pl.ds(pl.multiple_of((blk & 1) * 128, 128), 128)],
                inter_hbm.at[pl.ds(
                    pl.multiple_of(wid * _ACAP + blk * 128, 128), 128)])

        j1d[pl.ds(_ACAP - 16, 16)] = jnp.full((16,), dumpj, jnp.int32)
        for q in range(_ACAP // 128):
            qvec = jnp.full((16,), q, jnp.int32)
            for sv in range(8):
                plsc.store_scatter(
                    jrep, [qvec, sv * 16 + iota],
                    j1d[pl.ds(q * 128 + sv * 16, 16)])
        pltpu.sync_copy(jrep, jout_hbm.at[wid])

    return k1


def _build_k2():
    mesh = plsc.VectorSubcoreMesh(core_axis_name="c", subcore_axis_name="s")

    @functools.partial(
        pl.kernel,
        mesh=mesh,
        compiler_params=pltpu.CompilerParams(
            use_tc_tiling_on_sc=True, needs_layout_passes=False),
        out_type=jax.ShapeDtypeStruct((_B // 2, 128), jnp.float32),
        scratch_types=[
            pltpu.VMEM_SHARED(((_B // 2 + 16) * 8,), jnp.int32),
            pltpu.VMEM((_ACAP // 128, 128), jnp.int32),
            pltpu.VMEM((_ACAP // 128, 128), jnp.int32),
            pltpu.VMEM((_ACAP // 128, 128), jnp.int32),
            pltpu.VMEM((_CPW * 128 * 8,), jnp.int32),
            pltpu.VMEM((_CPW * 128,), jnp.int32),
            pltpu.VMEM((2, 128, 128), jnp.float32),
            pltpu.VMEM((64, 128), jnp.float32),
            pltpu.SemaphoreType.DMA,
            pltpu.SemaphoreType.DMA,
        ],
    )
    def k2(jin_hbm, inter_hbm, out_hbm, perm, jrow, jrow8, mval, m8, m1d,
           ring, pk, scsem, gsem):
        cid = lax.axis_index("c")
        sid = lax.axis_index("s")
        wid = sid * 2 + cid
        iota = _iota16()

        for t_local in range(2):
            t = sid * 2 + t_local
            pltpu.sync_copy(jin_hbm.at[t], jrow)

            def fill_body(q, _):
                qv = jnp.full((16,), q, jnp.int32)
                jpw = _CPW * 128
                for sv in range(8):
                    col = sv * 16 + iota
                    jv = plsc.load_gather(jrow, [qv, col])
                    blk = jv // jpw
                    keep = ((blk & 1) == cid) & (jv < _B)
                    slot = ((blk >> 1) * jpw + jv - blk * jpw) * 8
                    slot = jnp.where(keep, slot, (_B // 2 + sid) * 8)
                    plsc.store_scatter(jrow8, [qv, col], slot)
                    plsc.store_scatter(
                        mval, [qv, col], t * _ACAP + q * 128 + col)
                return _

            lax.fori_loop(0, _ACAP // 128, fill_body, 0)
            for q in range(_ACAP // 128):
                pltpu.async_copy(mval.at[q], perm.at[jrow8.at[q]], scsem)
            for q in range(_ACAP // 128):
                pltpu.make_async_copy(
                    mval.at[q], perm.at[jrow8.at[q]], scsem).wait()

        plsc.subcore_barrier()

        j0l = sid * (_CPW * 128)
        pltpu.sync_copy(
            perm.at[pl.ds(pl.multiple_of(j0l * 8, 128), _CPW * 128 * 8)], m8)

        def compact_body(v, _):
            m1d[pl.ds(pl.multiple_of(v * 16, 16), 16)] = plsc.load_gather(
                m8, [(v * 16 + iota) * 8])
            return _

        lax.fori_loop(0, _CPW * 8, compact_body, 0)

        pltpu.async_copy(
            inter_hbm.at[m1d.at[pl.ds(0, 128)]], ring.at[0], gsem)

        def chunk_body(i, carry):
            for pb in range(2):
                r = i * 2 + pb
                pltpu.make_async_copy(
                    inter_hbm.at[m1d.at[pl.ds(pl.multiple_of(r * 128, 128),
                                              128)]],
                    ring.at[pb], gsem).wait()

                @pl.when(r + 1 < _CPW)
                def _():
                    pltpu.async_copy(
                        inter_hbm.at[m1d.at[pl.ds(
                            pl.multiple_of((r + 1) * 128, 128), 128)]],
                        ring.at[1 - pb], gsem)

                pbv = jnp.full((16,), pb, jnp.int32)

                def pack_body(t, _):
                    tv = jnp.full((16,), t, jnp.int32)
                    for sv in range(4):
                        col = sv * 16 + iota
                        a = plsc.load_gather(ring, [pbv, tv * 2, col])
                        b = plsc.load_gather(ring, [pbv, tv * 2 + 1, col])
                        plsc.store_scatter(pk, [tv, col], a)
                        plsc.store_scatter(pk, [tv, 64 + col], b)
                    return _

                lax.fori_loop(0, 64, pack_body, 0)
                pltpu.sync_copy(
                    pk, out_hbm.at[pl.ds(
                        pl.multiple_of((wid * _CPW + r) * 64, 64), 64)])
            return carry

        lax.fori_loop(0, _CPW // 2, chunk_body, 0)

    return k2


def kernel(indices, table):
    idx = indices.reshape(-1).astype(jnp.int32)
    tp = table.T
    tail = jnp.pad(tp[:, _NCOLS - 64:], ((0, 0), (0, 64)))
    inter, jlists = _build_k1()(idx, tp, tail)
    out = _build_k2()(jlists, inter)
    return out.reshape(_BATCH, _N_FIELDS, _DIM)


# speedup vs baseline: 1.1289x; 1.1289x over previous
import functools

import jax
import jax.numpy as jnp
from jax import lax
from jax.experimental import pallas as pl
from jax.experimental.pallas import tpu as pltpu
from jax.experimental.pallas import tpu_sc as plsc

_N_FIELDS = 26
_BATCH = 4096
_VOCAB = 100000
_DIM = 64
_B = _N_FIELDS * _BATCH
_NW = 32
_NCOLS = _N_FIELDS * _VOCAB

_WIN = 256
_NWIN = (_NCOLS + _WIN - 1) // _WIN
_LAST_G = _NWIN - 1
_WPW = 318

_ACAP = 3840
_NBUCK = 20
_BCAP = 368
_SENT = 0x7FFFFFF0

_IDXC = 13
_IDXR = 64

_M_ROWS = _NW * _ACAP
_PERM_N = _B + 256

_CPW = _B // (_NW * 128)


def _iota16():
    return lax.broadcasted_iota(jnp.int32, (16,), 0)


def _popcount(mask):
    return plsc.all_reduce_population_count(mask)[0]


def _cstore(ref_a, xa, ref_b, xb, m, off, trash):
    rank = plsc.cumsum(m.astype(jnp.int32))
    pos = jnp.where(m, off + rank - 1, trash)
    plsc.store_scatter(ref_a, [pos], xa)
    if ref_b is not None:
        plsc.store_scatter(ref_b, [pos], xb)
    return rank[15]


def _build_k1():
    mesh = plsc.VectorSubcoreMesh(core_axis_name="c", subcore_axis_name="s")

    @functools.partial(
        pl.kernel,
        mesh=mesh,
        compiler_params=pltpu.CompilerParams(
            use_tc_tiling_on_sc=True, needs_layout_passes=False),
        out_type=(
            jax.ShapeDtypeStruct((_M_ROWS, 128), jnp.float32),
            jax.ShapeDtypeStruct((_NW, _ACAP // 128, 128), jnp.int32),
        ),
        scratch_types=[
            pltpu.VMEM((_IDXR * 128,), jnp.int32),
            pltpu.VMEM((_ACAP,), jnp.int32),
            pltpu.VMEM((_ACAP,), jnp.int32),
            pltpu.VMEM((_NBUCK * _BCAP,), jnp.int32),
            pltpu.VMEM((_NBUCK * _BCAP,), jnp.int32),
            pltpu.VMEM((64, _WIN), jnp.float32),
            pltpu.VMEM((64, _WIN), jnp.float32),
            pltpu.VMEM((64, _WIN), jnp.float32),
            pltpu.VMEM((256, 128), jnp.float32),
            pltpu.VMEM((64,), jnp.int32),
            pltpu.VMEM((_ACAP,), jnp.int32),
            pltpu.VMEM((_ACAP // 128, 128), jnp.int32),
            pltpu.SemaphoreType.DMA,
            pltpu.SemaphoreType.DMA,
        ],
    )
    def k1(idx_hbm, tp_hbm, tail_hbm, inter_hbm, jout_hbm,
           idxbuf0, rstage, jstage, rbuck, jbuck, slab0, slab1, slab2,
           outstage, winr, j1d, jrep, isem, ssem):
        wid = lax.axis_index("s") * 2 + lax.axis_index("c")
        lo = wid * (_WPW * _WIN)
        hi = jnp.minimum(lo + _WPW * _WIN, _NCOLS)
        dumpj = _B + wid
        iota = _iota16()
        slabs = (slab0, slab1, slab2)

        for v in range(_ACAP // 16):
            rstage[pl.ds(v * 16, 16)] = jnp.full((16,), _SENT, jnp.int32)
            j1d[pl.ds(v * 16, 16)] = jnp.full((16,), dumpj, jnp.int32)
        for v in range(_NBUCK * _BCAP // 16):
            rbuck[pl.ds(v * 16, 16)] = jnp.full((16,), _SENT, jnp.int32)
        for v in range(4):
            winr[pl.ds(v * 16, 16)] = jnp.zeros((16,), jnp.int32)

        def chunk_body(cc, cnt):
            src = idx_hbm.at[pl.ds(
                pl.multiple_of(cc * _IDXR * 128, 1024), _IDXR * 128)]
            pltpu.sync_copy(src, idxbuf0)

            def row_body(rr, cnt2):
                grow = cc * _IDXR + rr
                off = (grow // 32) * _VOCAB
                for sv in range(8):
                    r = idxbuf0[pl.ds(
                        pl.multiple_of(rr * 128 + sv * 16, 16), 16)] + off
                    m = (r >= lo) & (r < hi)
                    jv = grow * 128 + sv * 16 + iota
                    o = jnp.minimum(cnt2, _ACAP - 32)
                    nc = _cstore(rstage, r, jstage, jv, m, o, _ACAP - 1)
                    cnt2 = cnt2 + nc
                return cnt2

            return lax.fori_loop(0, _IDXR, row_body, cnt)

        cnt = lax.fori_loop(0, _IDXC, chunk_body, jnp.int32(0))
        rstage[pl.ds(_ACAP - 16, 16)] = jnp.full((16,), _SENT, jnp.int32)

        def buck_body(v, cnts):
            p16 = pl.multiple_of(v * 16, 16)
            r = rstage[pl.ds(p16, 16)]
            jv = jstage[pl.ds(p16, 16)]
            bid = (r - lo) >> 12
            new = []
            for b in range(_NBUCK):
                m = bid == b
                o = b * _BCAP + jnp.minimum(cnts[b], _BCAP - 32)
                nc = _cstore(rbuck, r, jbuck, jv, m, o, b * _BCAP + _BCAP - 1)
                new.append(cnts[b] + nc)
            return tuple(new)

        lax.fori_loop(0, _ACAP // 16, buck_body,
                      tuple(jnp.int32(0) for _ in range(_NBUCK)))
        for b in range(_NBUCK):
            rbuck[pl.ds(b * _BCAP + _BCAP - 16, 16)] = jnp.full(
                (16,), _SENT, jnp.int32)

        def issue_slab(k_w, pb):
            g = wid * _WPW + k_w

            @pl.when(g < _LAST_G)
            def _():
                pltpu.async_copy(
                    tp_hbm.at[:, pl.ds(pl.multiple_of(g * _WIN, _WIN), _WIN)],
                    slabs[pb], ssem)

            @pl.when(g == _LAST_G)
            def _():
                pltpu.async_copy(
                    tail_hbm, slabs[pb].at[:, pl.ds(0, 128)], ssem)

        def wait_slab(k_w, pb):
            g = wid * _WPW + k_w

            @pl.when(g < _LAST_G)
            def _():
                pltpu.make_async_copy(
                    tp_hbm.at[:, pl.ds(pl.multiple_of(g * _WIN, _WIN), _WIN)],
                    slabs[pb], ssem).wait()

            @pl.when(g == _LAST_G)
            def _():
                pltpu.make_async_copy(
                    tail_hbm, slabs[pb].at[:, pl.ds(0, 128)], ssem).wait()

        @pl.when(wid * _WPW < _NWIN)
        def _():
            issue_slab(0, 0)

        @pl.when(wid * _WPW + 1 < _NWIN)
        def _():
            issue_slab(1, 1)

        def win_body(k3, base):
            for pb in range(3):
                k_w = k3 * 3 + pb
                g = wid * _WPW + k_w
                wait_slab(k_w, pb)

                @pl.when(k_w + 2 < _WPW)
                def _():
                    @pl.when(g + 2 < _NWIN)
                    def _():
                        issue_slab(k_w + 2, (pb + 2) % 3)

                bk = k_w >> 4

                def scan_body(v, wcnt):
                    p = pl.multiple_of(bk * _BCAP + v * 16, 16)
                    r = rbuck[pl.ds(p, 16)]
                    jv = jbuck[pl.ds(p, 16)]
                    m = (r >> 8) == g
                    rcol = r & 255
                    o = jnp.minimum(wcnt, 47)
                    nc = _cstore(winr, rcol, None, None, m, o, 63)
                    ob = jnp.minimum(base + wcnt, _ACAP - 32)
                    _cstore(j1d, jv, None, None, m, ob, _ACAP - 1)
                    return wcnt + nc

                wcnt = lax.fori_loop(0, _BCAP // 16, scan_body, jnp.int32(0))

                def ext_body(gi, _):
                    rcol = winr[pl.ds(pl.multiple_of(gi * 16, 16), 16)]
                    rows = (base + gi * 16 + iota) & 255
                    for d in range(64):
                        dvec = jnp.full((16,), d, jnp.int32)
                        vals = plsc.load_gather(slabs[pb], [dvec, rcol])
                        plsc.store_scatter(outstage, [rows, dvec], vals)
                    return _

                lax.fori_loop(0, (wcnt + 15) >> 4, ext_body, 0)

                nbase = base + wcnt

                @pl.when((nbase >> 7) > (base >> 7))
                def _():
                    blk = base >> 7
                    pltpu.sync_copy(
                        outstage.at[pl.ds(
                            pl.multiple_of((blk & 1) * 128, 128), 128)],
                        inter_hbm.at[pl.ds(
                            pl.multiple_of(wid * _ACAP + blk * 128, 128),
                            128)])

                base = nbase
            return base

        base = lax.fori_loop(0, _WPW // 3, win_body, jnp.int32(0))

        @pl.when((base & 127) != 0)
        def _():
            blk = base >> 7
            pltpu.sync_copy(
                outstage.at[pl.ds(pl.multiple_of((blk & 1) * 128, 128), 128)],
                inter_hbm.at[pl.ds(
                    pl.multiple_of(wid * _ACAP + blk * 128, 128), 128)])

        j1d[pl.ds(_ACAP - 16, 16)] = jnp.full((16,), dumpj, jnp.int32)
        for q in range(_ACAP // 128):
            qvec = jnp.full((16,), q, jnp.int32)
            for sv in range(8):
                plsc.store_scatter(
                    jrep, [qvec, sv * 16 + iota],
                    j1d[pl.ds(q * 128 + sv * 16, 16)])
        pltpu.sync_copy(jrep, jout_hbm.at[wid])

    return k1


def _build_k2():
    mesh = plsc.VectorSubcoreMesh(core_axis_name="c", subcore_axis_name="s")

    @functools.partial(
        pl.kernel,
        mesh=mesh,
        compiler_params=pltpu.CompilerParams(
            use_tc_tiling_on_sc=True, needs_layout_passes=False),
        out_type=jax.ShapeDtypeStruct((_B // 2, 128), jnp.float32),
        scratch_types=[
            pltpu.VMEM_SHARED(((_B // 2 + 16) * 8,), jnp.int32),
            pltpu.VMEM((_ACAP // 128, 128), jnp.int32),
            pltpu.VMEM((_ACAP // 128, 128), jnp.int32),
            pltpu.VMEM((_ACAP // 128, 128), jnp.int32),
            pltpu.VMEM((_CPW * 128 * 8,), jnp.int32),
            pltpu.VMEM((_CPW * 128,), jnp.int32),
            pltpu.VMEM((2, 128, 128), jnp.float32),
            pltpu.VMEM((64, 128), jnp.float32),
            pltpu.SemaphoreType.DMA,
            pltpu.SemaphoreType.DMA,
        ],
    )
    def k2(jin_hbm, inter_hbm, out_hbm, perm, jrow, jrow8, mval, m8, m1d,
           ring, pk, scsem, gsem):
        cid = lax.axis_index("c")
        sid = lax.axis_index("s")
        wid = sid * 2 + cid
        iota = _iota16()

        for t_local in range(2):
            t = sid * 2 + t_local
            pltpu.sync_copy(jin_hbm.at[t], jrow)

            def fill_body(q, _):
                qv = jnp.full((16,), q, jnp.int32)
                jpw = _CPW * 128
                for sv in range(8):
                    col = sv * 16 + iota
                    jv = plsc.load_gather(jrow, [qv, col])
                    blk = jv // jpw
                    keep = ((blk & 1) == cid) & (jv < _B)
                    slot = ((blk >> 1) * jpw + jv - blk * jpw) * 8
                    slot = jnp.where(keep, slot, (_B // 2 + sid) * 8)
                    plsc.store_scatter(jrow8, [qv, col], slot)
                    plsc.store_scatter(
                        mval, [qv, col], t * _ACAP + q * 128 + col)
                return _

            lax.fori_loop(0, _ACAP // 128, fill_body, 0)
            for q in range(_ACAP // 128):
                pltpu.async_copy(mval.at[q], perm.at[jrow8.at[q]], scsem)
            for q in range(_ACAP // 128):
                pltpu.make_async_copy(
                    mval.at[q], perm.at[jrow8.at[q]], scsem).wait()

        plsc.subcore_barrier()

        j0l = sid * (_CPW * 128)
        pltpu.sync_copy(
            perm.at[pl.ds(pl.multiple_of(j0l * 8, 128), _CPW * 128 * 8)], m8)

        def compact_body(v, _):
            m1d[pl.ds(pl.multiple_of(v * 16, 16), 16)] = plsc.load_gather(
                m8, [(v * 16 + iota) * 8])
            return _

        lax.fori_loop(0, _CPW * 8, compact_body, 0)

        pltpu.async_copy(
            inter_hbm.at[m1d.at[pl.ds(0, 128)]], ring.at[0], gsem)

        def chunk_body(i, carry):
            for pb in range(2):
                r = i * 2 + pb
                pltpu.make_async_copy(
                    inter_hbm.at[m1d.at[pl.ds(pl.multiple_of(r * 128, 128),
                                              128)]],
                    ring.at[pb], gsem).wait()

                @pl.when(r + 1 < _CPW)
                def _():
                    pltpu.async_copy(
                        inter_hbm.at[m1d.at[pl.ds(
                            pl.multiple_of((r + 1) * 128, 128), 128)]],
                        ring.at[1 - pb], gsem)

                pbv = jnp.full((16,), pb, jnp.int32)

                def pack_body(t, _):
                    tv = jnp.full((16,), t, jnp.int32)
                    for sv in range(4):
                        col = sv * 16 + iota
                        a = plsc.load_gather(ring, [pbv, tv * 2, col])
                        b = plsc.load_gather(ring, [pbv, tv * 2 + 1, col])
                        plsc.store_scatter(pk, [tv, col], a)
                        plsc.store_scatter(pk, [tv, 64 + col], b)
                    return _

                lax.fori_loop(0, 64, pack_body, 0)
                pltpu.sync_copy(
                    pk, out_hbm.at[pl.ds(
                        pl.multiple_of((wid * _CPW + r) * 64, 64), 64)])
            return carry

        lax.fori_loop(0, _CPW // 2, chunk_body, 0)

    return k2


def kernel(indices, table):
    idx = indices.reshape(-1).astype(jnp.int32)
    tp = table.T
    tail = jnp.pad(tp[:, _NCOLS - 64:], ((0, 0), (0, 64)))
    inter, jlists = _build_k1()(idx, tp, tail)
    out = _build_k2()(jlists, inter)
    return out.reshape(_BATCH, _N_FIELDS, _DIM)


# speedup vs baseline: 1.1606x; 1.0281x over previous
import functools

import jax
import jax.numpy as jnp
from jax import lax
from jax.experimental import pallas as pl
from jax.experimental.pallas import tpu as pltpu
from jax.experimental.pallas import tpu_sc as plsc

_N_FIELDS = 26
_BATCH = 4096
_VOCAB = 100000
_DIM = 64
_B = _N_FIELDS * _BATCH
_NW = 32
_NCOLS = _N_FIELDS * _VOCAB

_WIN = 256
_NWIN = (_NCOLS + _WIN - 1) // _WIN
_LAST_G = _NWIN - 1
_WPW = 318

_ACAP = 3840
_NBUCK = 20
_BCAP = 368
_SENT = 0x7FFFFFF0

_IDXC = 13
_IDXR = 64

_M_ROWS = _NW * _ACAP
_PERM_N = _B + 256

_CPW = _B // (_NW * 128)


def _iota16():
    return lax.broadcasted_iota(jnp.int32, (16,), 0)


def _popcount(mask):
    return plsc.all_reduce_population_count(mask)[0]


def _cstore(ref_a, xa, ref_b, xb, m, off, trash):
    rank = plsc.cumsum(m.astype(jnp.int32))
    pos = jnp.where(m, off + rank - 1, trash)
    plsc.store_scatter(ref_a, [pos], xa)
    if ref_b is not None:
        plsc.store_scatter(ref_b, [pos], xb)
    return rank[15]


def _build_k1():
    mesh = plsc.VectorSubcoreMesh(core_axis_name="c", subcore_axis_name="s")

    @functools.partial(
        pl.kernel,
        mesh=mesh,
        compiler_params=pltpu.CompilerParams(
            use_tc_tiling_on_sc=True, needs_layout_passes=False),
        out_type=(
            jax.ShapeDtypeStruct((_M_ROWS, 128), jnp.float32),
            jax.ShapeDtypeStruct((_NW, _ACAP // 128, 128), jnp.int32),
        ),
        scratch_types=[
            pltpu.VMEM((_IDXR * 128,), jnp.int32),
            pltpu.VMEM((_ACAP,), jnp.int32),
            pltpu.VMEM((_ACAP,), jnp.int32),
            pltpu.VMEM((_NBUCK * _BCAP,), jnp.int32),
            pltpu.VMEM((_NBUCK * _BCAP,), jnp.int32),
            pltpu.VMEM((64, _WIN), jnp.float32),
            pltpu.VMEM((64, _WIN), jnp.float32),
            pltpu.VMEM((64, _WIN), jnp.float32),
            pltpu.VMEM((256, 128), jnp.float32),
            pltpu.VMEM((64,), jnp.int32),
            pltpu.VMEM((_ACAP,), jnp.int32),
            pltpu.VMEM((_ACAP // 128, 128), jnp.int32),
            pltpu.SemaphoreType.DMA,
            pltpu.SemaphoreType.DMA,
            pltpu.SemaphoreType.DMA,
        ],
    )
    def k1(idx_hbm, tp_hbm, tail_hbm, inter_hbm, jout_hbm,
           idxbuf0, rstage, jstage, rbuck, jbuck, slab0, slab1, slab2,
           outstage, winr, j1d, jrep, isem, ssem, fsem):
        wid = lax.axis_index("s") * 2 + lax.axis_index("c")
        lo = wid * (_WPW * _WIN)
        hi = jnp.minimum(lo + _WPW * _WIN, _NCOLS)
        dumpj = _B + wid
        iota = _iota16()
        slabs = (slab0, slab1, slab2)

        for v in range(_ACAP // 16):
            rstage[pl.ds(v * 16, 16)] = jnp.full((16,), _SENT, jnp.int32)
            j1d[pl.ds(v * 16, 16)] = jnp.full((16,), dumpj, jnp.int32)
        for v in range(_NBUCK * _BCAP // 16):
            rbuck[pl.ds(v * 16, 16)] = jnp.full((16,), _SENT, jnp.int32)
        for v in range(4):
            winr[pl.ds(v * 16, 16)] = jnp.zeros((16,), jnp.int32)

        def chunk_body(cc, cnt):
            src = idx_hbm.at[pl.ds(
                pl.multiple_of(cc * _IDXR * 128, 1024), _IDXR * 128)]
            pltpu.sync_copy(src, idxbuf0)

            def row_body(rr, cnt2):
                grow = cc * _IDXR + rr
                off = (grow // 32) * _VOCAB
                for sv in range(8):
                    r = idxbuf0[pl.ds(
                        pl.multiple_of(rr * 128 + sv * 16, 16), 16)] + off
                    m = (r >= lo) & (r < hi)
                    jv = grow * 128 + sv * 16 + iota
                    o = jnp.minimum(cnt2, _ACAP - 32)
                    nc = _cstore(rstage, r, jstage, jv, m, o, _ACAP - 1)
                    cnt2 = cnt2 + nc
                return cnt2

            return lax.fori_loop(0, _IDXR, row_body, cnt)

        cnt = lax.fori_loop(0, _IDXC, chunk_body, jnp.int32(0))
        rstage[pl.ds(_ACAP - 16, 16)] = jnp.full((16,), _SENT, jnp.int32)

        def buck_body(v, cnts):
            p16 = pl.multiple_of(v * 16, 16)
            r = rstage[pl.ds(p16, 16)]
            jv = jstage[pl.ds(p16, 16)]
            bid = (r - lo) >> 12
            new = []
            for b in range(_NBUCK):
                m = bid == b
                o = b * _BCAP + jnp.minimum(cnts[b], _BCAP - 32)
                nc = _cstore(rbuck, r, jbuck, jv, m, o, b * _BCAP + _BCAP - 1)
                new.append(cnts[b] + nc)
            return tuple(new)

        lax.fori_loop(0, _ACAP // 16, buck_body,
                      tuple(jnp.int32(0) for _ in range(_NBUCK)))
        for b in range(_NBUCK):
            rbuck[pl.ds(b * _BCAP + _BCAP - 16, 16)] = jnp.full(
                (16,), _SENT, jnp.int32)

        def issue_slab(k_w, pb):
            g = wid * _WPW + k_w

            @pl.when(g < _LAST_G)
            def _():
                pltpu.async_copy(
                    tp_hbm.at[:, pl.ds(pl.multiple_of(g * _WIN, _WIN), _WIN)],
                    slabs[pb], ssem)

            @pl.when(g == _LAST_G)
            def _():
                pltpu.async_copy(
                    tail_hbm, slabs[pb].at[:, pl.ds(0, 128)], ssem)

        def wait_slab(k_w, pb):
            g = wid * _WPW + k_w

            @pl.when(g < _LAST_G)
            def _():
                pltpu.make_async_copy(
                    tp_hbm.at[:, pl.ds(pl.multiple_of(g * _WIN, _WIN), _WIN)],
                    slabs[pb], ssem).wait()

            @pl.when(g == _LAST_G)
            def _():
                pltpu.make_async_copy(
                    tail_hbm, slabs[pb].at[:, pl.ds(0, 128)], ssem).wait()

        @pl.when(wid * _WPW < _NWIN)
        def _():
            issue_slab(0, 0)

        @pl.when(wid * _WPW + 1 < _NWIN)
        def _():
            issue_slab(1, 1)

        def win_body(k3, base):
            for pb in range(3):
                k_w = k3 * 3 + pb
                g = wid * _WPW + k_w
                wait_slab(k_w, pb)

                @pl.when(k_w + 2 < _WPW)
                def _():
                    @pl.when(g + 2 < _NWIN)
                    def _():
                        issue_slab(k_w + 2, (pb + 2) % 3)

                bk = k_w >> 4

                def scan_body(v, wcnt):
                    p = pl.multiple_of(bk * _BCAP + v * 16, 16)
                    r = rbuck[pl.ds(p, 16)]
                    jv = jbuck[pl.ds(p, 16)]
                    m = (r >> 8) == g
                    rcol = r & 255
                    o = jnp.minimum(wcnt, 47)
                    nc = _cstore(winr, rcol, None, None, m, o, 63)
                    ob = jnp.minimum(base + wcnt, _ACAP - 32)
                    _cstore(j1d, jv, None, None, m, ob, _ACAP - 1)
                    return wcnt + nc

                wcnt = lax.fori_loop(0, _BCAP // 16, scan_body, jnp.int32(0))
                nbase = base + wcnt
                crossing = (nbase >> 7) > (base >> 7)

                @pl.when(crossing & ((base >> 7) >= 1))
                def _():
                    pblk = (base >> 7) - 1
                    pltpu.make_async_copy(
                        outstage.at[pl.ds(
                            pl.multiple_of((pblk & 1) * 128, 128), 128)],
                        inter_hbm.at[pl.ds(
                            pl.multiple_of(wid * _ACAP + pblk * 128, 128),
                            128)], fsem).wait()

                def ext_body(gi, _):
                    rcol = winr[pl.ds(pl.multiple_of(gi * 16, 16), 16)]
                    rows = (base + gi * 16 + iota) & 255
                    for d in range(64):
                        dvec = jnp.full((16,), d, jnp.int32)
                        vals = plsc.load_gather(slabs[pb], [dvec, rcol])
                        plsc.store_scatter(outstage, [rows, dvec], vals)
                    return _

                lax.fori_loop(0, (wcnt + 15) >> 4, ext_body, 0)

                @pl.when(crossing)
                def _():
                    blk = base >> 7
                    pltpu.async_copy(
                        outstage.at[pl.ds(
                            pl.multiple_of((blk & 1) * 128, 128), 128)],
                        inter_hbm.at[pl.ds(
                            pl.multiple_of(wid * _ACAP + blk * 128, 128),
                            128)], fsem)

                base = nbase
            return base

        base = lax.fori_loop(0, _WPW // 3, win_body, jnp.int32(0))

        @pl.when((base >> 7) >= 1)
        def _():
            pblk = (base >> 7) - 1
            pltpu.make_async_copy(
                outstage.at[pl.ds(
                    pl.multiple_of((pblk & 1) * 128, 128), 128)],
                inter_hbm.at[pl.ds(
                    pl.multiple_of(wid * _ACAP + pblk * 128, 128), 128)],
                fsem).wait()

        @pl.when((base & 127) != 0)
        def _():
            blk = base >> 7
            pltpu.sync_copy(
                outstage.at[pl.ds(pl.multiple_of((blk & 1) * 128, 128), 128)],
                inter_hbm.at[pl.ds(
                    pl.multiple_of(wid * _ACAP + blk * 128, 128), 128)])

        j1d[pl.ds(_ACAP - 16, 16)] = jnp.full((16,), dumpj, jnp.int32)
        for q in range(_ACAP // 128):
            qvec = jnp.full((16,), q, jnp.int32)
            for sv in range(8):
                plsc.store_scatter(
                    jrep, [qvec, sv * 16 + iota],
                    j1d[pl.ds(q * 128 + sv * 16, 16)])
        pltpu.sync_copy(jrep, jout_hbm.at[wid])

    return k1


def _build_k2():
    mesh = plsc.VectorSubcoreMesh(core_axis_name="c", subcore_axis_name="s")

    @functools.partial(
        pl.kernel,
        mesh=mesh,
        compiler_params=pltpu.CompilerParams(
            use_tc_tiling_on_sc=True, needs_layout_passes=False),
        out_type=jax.ShapeDtypeStruct((_B // 2, 128), jnp.float32),
        scratch_types=[
            pltpu.VMEM_SHARED(((_B // 2 + 16) * 8,), jnp.int32),
            pltpu.VMEM((_ACAP // 128, 128), jnp.int32),
            pltpu.VMEM((_ACAP // 128, 128), jnp.int32),
            pltpu.VMEM((_ACAP // 128, 128), jnp.int32),
            pltpu.VMEM((_CPW * 128 * 8,), jnp.int32),
            pltpu.VMEM((_CPW * 128,), jnp.int32),
            pltpu.VMEM((2, 128, 128), jnp.float32),
            pltpu.VMEM((64, 128), jnp.float32),
            pltpu.SemaphoreType.DMA,
            pltpu.SemaphoreType.DMA,
        ],
    )
    def k2(jin_hbm, inter_hbm, out_hbm, perm, jrow, jrow8, mval, m8, m1d,
           ring, pk, scsem, gsem):
        cid = lax.axis_index("c")
        sid = lax.axis_index("s")
        wid = sid * 2 + cid
        iota = _iota16()

        for t_local in range(2):
            t = sid * 2 + t_local
            pltpu.sync_copy(jin_hbm.at[t], jrow)

            def fill_body(q, _):
                qv = jnp.full((16,), q, jnp.int32)
                jpw = _CPW * 128
                for sv in range(8):
                    col = sv * 16 + iota
                    jv = plsc.load_gather(jrow, [qv, col])
                    blk = jv // jpw
                    keep = ((blk & 1) == cid) & (jv < _B)
                    slot = ((blk >> 1) * jpw + jv - blk * jpw) * 8
                    slot = jnp.where(keep, slot, (_B // 2 + sid) * 8)
                    plsc.store_scatter(jrow8, [qv, col], slot)
                    plsc.store_scatter(
                        mval, [qv, col], t * _ACAP + q * 128 + col)
                return _

            lax.fori_loop(0, _ACAP // 128, fill_body, 0)
            for q in range(_ACAP // 128):
                pltpu.async_copy(mval.at[q], perm.at[jrow8.at[q]], scsem)
            for q in range(_ACAP // 128):
                pltpu.make_async_copy(
                    mval.at[q], perm.at[jrow8.at[q]], scsem).wait()

        plsc.subcore_barrier()

        j0l = sid * (_CPW * 128)
        pltpu.sync_copy(
            perm.at[pl.ds(pl.multiple_of(j0l * 8, 128), _CPW * 128 * 8)], m8)

        def compact_body(v, _):
            m1d[pl.ds(pl.multiple_of(v * 16, 16), 16)] = plsc.load_gather(
                m8, [(v * 16 + iota) * 8])
            return _

        lax.fori_loop(0, _CPW * 8, compact_body, 0)

        pltpu.async_copy(
            inter_hbm.at[m1d.at[pl.ds(0, 128)]], ring.at[0], gsem)

        def chunk_body(i, carry):
            for pb in range(2):
                r = i * 2 + pb
                pltpu.make_async_copy(
                    inter_hbm.at[m1d.at[pl.ds(pl.multiple_of(r * 128, 128),
                                              128)]],
                    ring.at[pb], gsem).wait()

                @pl.when(r + 1 < _CPW)
                def _():
                    pltpu.async_copy(
                        inter_hbm.at[m1d.at[pl.ds(
                            pl.multiple_of((r + 1) * 128, 128), 128)]],
                        ring.at[1 - pb], gsem)

                pbv = jnp.full((16,), pb, jnp.int32)

                def pack_body(t, _):
                    tv = jnp.full((16,), t, jnp.int32)
                    for sv in range(4):
                        col = sv * 16 + iota
                        a = plsc.load_gather(ring, [pbv, tv * 2, col])
                        b = plsc.load_gather(ring, [pbv, tv * 2 + 1, col])
                        plsc.store_scatter(pk, [tv, col], a)
                        plsc.store_scatter(pk, [tv, 64 + col], b)
                    return _

                lax.fori_loop(0, 64, pack_body, 0)
                pltpu.sync_copy(
                    pk, out_hbm.at[pl.ds(
                        pl.multiple_of((wid * _CPW + r) * 64, 64), 64)])
            return carry

        lax.fori_loop(0, _CPW // 2, chunk_body, 0)

    return k2


def kernel(indices, table):
    idx = indices.reshape(-1).astype(jnp.int32)
    tp = table.T
    tail = jnp.pad(tp[:, _NCOLS - 64:], ((0, 0), (0, 64)))
    inter, jlists = _build_k1()(idx, tp, tail)
    out = _build_k2()(jlists, inter)
    return out.reshape(_BATCH, _N_FIELDS, _DIM)


# speedup vs baseline: 1.1806x; 1.0172x over previous
import functools

import jax
import jax.numpy as jnp
from jax import lax
from jax.experimental import pallas as pl
from jax.experimental.pallas import tpu as pltpu
from jax.experimental.pallas import tpu_sc as plsc

_N_FIELDS = 26
_BATCH = 4096
_VOCAB = 100000
_DIM = 64
_B = _N_FIELDS * _BATCH
_NW = 32
_NCOLS = _N_FIELDS * _VOCAB

_WIN = 256
_NWIN = (_NCOLS + _WIN - 1) // _WIN
_LAST_G = _NWIN - 1
_WPW = 318

_ACAP = 3840
_NBUCK = 20
_BCAP = 368
_SENT = 0x7FFFFFF0

_IDXC = 13
_IDXR = 64

_M_ROWS = _NW * _ACAP
_PERM_N = _B + 256

_CPW = _B // (_NW * 128)


def _iota16():
    return lax.broadcasted_iota(jnp.int32, (16,), 0)


def _popcount(mask):
    return plsc.all_reduce_population_count(mask)[0]


def _cstore(ref_a, xa, ref_b, xb, m, off, trash):
    rank = plsc.cumsum(m.astype(jnp.int32))
    pos = jnp.where(m, off + rank - 1, trash)
    plsc.store_scatter(ref_a, [pos], xa)
    if ref_b is not None:
        plsc.store_scatter(ref_b, [pos], xb)
    return rank[15]


def _build_k1():
    mesh = plsc.VectorSubcoreMesh(core_axis_name="c", subcore_axis_name="s")

    @functools.partial(
        pl.kernel,
        mesh=mesh,
        compiler_params=pltpu.CompilerParams(
            use_tc_tiling_on_sc=True, needs_layout_passes=False),
        out_type=(
            jax.ShapeDtypeStruct((_M_ROWS, 128), jnp.float32),
            jax.ShapeDtypeStruct((_NW, _ACAP // 128, 128), jnp.int32),
        ),
        scratch_types=[
            pltpu.VMEM((_IDXR * 128,), jnp.int32),
            pltpu.VMEM((_IDXR * 128,), jnp.int32),
            pltpu.VMEM((_ACAP,), jnp.int32),
            pltpu.VMEM((_ACAP,), jnp.int32),
            pltpu.VMEM((_NBUCK * _BCAP,), jnp.int32),
            pltpu.VMEM((_NBUCK * _BCAP,), jnp.int32),
            pltpu.VMEM((64, _WIN), jnp.float32),
            pltpu.VMEM((64, _WIN), jnp.float32),
            pltpu.VMEM((64, _WIN), jnp.float32),
            pltpu.VMEM((256, 128), jnp.float32),
            pltpu.VMEM((64,), jnp.int32),
            pltpu.VMEM((_ACAP,), jnp.int32),
            pltpu.VMEM((_ACAP // 128, 128), jnp.int32),
            pltpu.SemaphoreType.DMA,
            pltpu.SemaphoreType.DMA,
            pltpu.SemaphoreType.DMA,
        ],
    )
    def k1(idx_hbm, tp_hbm, tail_hbm, inter_hbm, jout_hbm,
           idxbuf0, idxbuf1, rstage, jstage, rbuck, jbuck, slab0, slab1, slab2,
           outstage, winr, j1d, jrep, isem, ssem, fsem):
        wid = lax.axis_index("s") * 2 + lax.axis_index("c")
        lo = wid * (_WPW * _WIN)
        hi = jnp.minimum(lo + _WPW * _WIN, _NCOLS)
        dumpj = _B + wid
        iota = _iota16()
        slabs = (slab0, slab1, slab2)

        for v in range(_ACAP // 16):
            rstage[pl.ds(v * 16, 16)] = jnp.full((16,), _SENT, jnp.int32)
            j1d[pl.ds(v * 16, 16)] = jnp.full((16,), dumpj, jnp.int32)
        for v in range(_NBUCK * _BCAP // 16):
            rbuck[pl.ds(v * 16, 16)] = jnp.full((16,), _SENT, jnp.int32)
        for v in range(4):
            winr[pl.ds(v * 16, 16)] = jnp.zeros((16,), jnp.int32)

        idxbufs = (idxbuf0, idxbuf1)
        pltpu.async_copy(idx_hbm.at[pl.ds(0, _IDXR * 128)], idxbuf0, isem)

        def chunk_body(cc, cnt):
            for pb in range(2):
                c2 = cc * 2 + pb
                src = idx_hbm.at[pl.ds(
                    pl.multiple_of(c2 * _IDXR * 128, 1024), _IDXR * 128)]
                pltpu.make_async_copy(src, idxbufs[pb], isem).wait()

                @pl.when(c2 + 1 < _IDXC)
                def _():
                    nsrc = idx_hbm.at[pl.ds(
                        pl.multiple_of((c2 + 1) * _IDXR * 128, 1024),
                        _IDXR * 128)]
                    pltpu.async_copy(nsrc, idxbufs[1 - pb], isem)

                def row_body(rr, cnt2):
                    grow = c2 * _IDXR + rr
                    off = (grow // 32) * _VOCAB
                    for sv in range(8):
                        r = idxbufs[pb][pl.ds(
                            pl.multiple_of(rr * 128 + sv * 16, 16), 16)] + off
                        m = (r >= lo) & (r < hi)
                        jv = grow * 128 + sv * 16 + iota
                        o = jnp.minimum(cnt2, _ACAP - 32)
                        nc = _cstore(rstage, r, jstage, jv, m, o, _ACAP - 1)
                        cnt2 = cnt2 + nc
                    return cnt2

                cnt = lax.fori_loop(0, _IDXR, row_body, cnt)
            return cnt

        cnt = lax.fori_loop(0, _IDXC // 2, chunk_body, jnp.int32(0))

        src12 = idx_hbm.at[pl.ds(12 * _IDXR * 128, _IDXR * 128)]
        pltpu.make_async_copy(src12, idxbuf0, isem).wait()

        def row_body12(rr, cnt2):
            grow = 12 * _IDXR + rr
            off = (grow // 32) * _VOCAB
            for sv in range(8):
                r = idxbuf0[pl.ds(
                    pl.multiple_of(rr * 128 + sv * 16, 16), 16)] + off
                m = (r >= lo) & (r < hi)
                jv = grow * 128 + sv * 16 + iota
                o = jnp.minimum(cnt2, _ACAP - 32)
                nc = _cstore(rstage, r, jstage, jv, m, o, _ACAP - 1)
                cnt2 = cnt2 + nc
            return cnt2

        cnt = lax.fori_loop(0, _IDXR, row_body12, cnt)
        rstage[pl.ds(_ACAP - 16, 16)] = jnp.full((16,), _SENT, jnp.int32)

        def buck_body(v, cnts):
            p16 = pl.multiple_of(v * 16, 16)
            r = rstage[pl.ds(p16, 16)]
            jv = jstage[pl.ds(p16, 16)]
            bid = (r - lo) >> 12
            new = []
            for b in range(_NBUCK):
                m = bid == b
                o = b * _BCAP + jnp.minimum(cnts[b], _BCAP - 32)
                nc = _cstore(rbuck, r, jbuck, jv, m, o, b * _BCAP + _BCAP - 1)
                new.append(cnts[b] + nc)
            return tuple(new)

        lax.fori_loop(0, _ACAP // 16, buck_body,
                      tuple(jnp.int32(0) for _ in range(_NBUCK)))
        for b in range(_NBUCK):
            rbuck[pl.ds(b * _BCAP + _BCAP - 16, 16)] = jnp.full(
                (16,), _SENT, jnp.int32)

        def issue_slab(k_w, pb):
            g = wid * _WPW + k_w

            @pl.when(g < _LAST_G)
            def _():
                pltpu.async_copy(
                    tp_hbm.at[:, pl.ds(pl.multiple_of(g * _WIN, _WIN), _WIN)],
                    slabs[pb], ssem)

            @pl.when(g == _LAST_G)
            def _():
                pltpu.async_copy(
                    tail_hbm, slabs[pb].at[:, pl.ds(0, 128)], ssem)

        def wait_slab(k_w, pb):
            g = wid * _WPW + k_w

            @pl.when(g < _LAST_G)
            def _():
                pltpu.make_async_copy(
                    tp_hbm.at[:, pl.ds(pl.multiple_of(g * _WIN, _WIN), _WIN)],
                    slabs[pb], ssem).wait()

            @pl.when(g == _LAST_G)
            def _():
                pltpu.make_async_copy(
                    tail_hbm, slabs[pb].at[:, pl.ds(0, 128)], ssem).wait()

        @pl.when(wid * _WPW < _NWIN)
        def _():
            issue_slab(0, 0)

        @pl.when(wid * _WPW + 1 < _NWIN)
        def _():
            issue_slab(1, 1)

        def win_body(k3, base):
            for pb in range(3):
                k_w = k3 * 3 + pb
                g = wid * _WPW + k_w
                wait_slab(k_w, pb)

                @pl.when(k_w + 2 < _WPW)
                def _():
                    @pl.when(g + 2 < _NWIN)
                    def _():
                        issue_slab(k_w + 2, (pb + 2) % 3)

                bk = k_w >> 4

                def scan_body(v, wcnt):
                    p = pl.multiple_of(bk * _BCAP + v * 16, 16)
                    r = rbuck[pl.ds(p, 16)]
                    jv = jbuck[pl.ds(p, 16)]
                    m = (r >> 8) == g
                    rcol = r & 255
                    o = jnp.minimum(wcnt, 47)
                    nc = _cstore(winr, rcol, None, None, m, o, 63)
                    ob = jnp.minimum(base + wcnt, _ACAP - 32)
                    _cstore(j1d, jv, None, None, m, ob, _ACAP - 1)
                    return wcnt + nc

                wcnt = lax.fori_loop(0, _BCAP // 16, scan_body, jnp.int32(0))
                nbase = base + wcnt
                crossing = (nbase >> 7) > (base >> 7)

                @pl.when(crossing & ((base >> 7) >= 1))
                def _():
                    pblk = (base >> 7) - 1
                    pltpu.make_async_copy(
                        outstage.at[pl.ds(
                            pl.multiple_of((pblk & 1) * 128, 128), 128)],
                        inter_hbm.at[pl.ds(
                            pl.multiple_of(wid * _ACAP + pblk * 128, 128),
                            128)], fsem).wait()

                def ext_body(gi, _):
                    rcol = winr[pl.ds(pl.multiple_of(gi * 16, 16), 16)]
                    rows = (base + gi * 16 + iota) & 255
                    for d in range(64):
                        dvec = jnp.full((16,), d, jnp.int32)
                        vals = plsc.load_gather(slabs[pb], [dvec, rcol])
                        plsc.store_scatter(outstage, [rows, dvec], vals)
                    return _

                lax.fori_loop(0, (wcnt + 15) >> 4, ext_body, 0)

                @pl.when(crossing)
                def _():
                    blk = base >> 7
                    pltpu.async_copy(
                        outstage.at[pl.ds(
                            pl.multiple_of((blk & 1) * 128, 128), 128)],
                        inter_hbm.at[pl.ds(
                            pl.multiple_of(wid * _ACAP + blk * 128, 128),
                            128)], fsem)

                base = nbase
            return base

        base = lax.fori_loop(0, _WPW // 3, win_body, jnp.int32(0))

        @pl.when((base >> 7) >= 1)
        def _():
            pblk = (base >> 7) - 1
            pltpu.make_async_copy(
                outstage.at[pl.ds(
                    pl.multiple_of((pblk & 1) * 128, 128), 128)],
                inter_hbm.at[pl.ds(
                    pl.multiple_of(wid * _ACAP + pblk * 128, 128), 128)],
                fsem).wait()

        @pl.when((base & 127) != 0)
        def _():
            blk = base >> 7
            pltpu.sync_copy(
                outstage.at[pl.ds(pl.multiple_of((blk & 1) * 128, 128), 128)],
                inter_hbm.at[pl.ds(
                    pl.multiple_of(wid * _ACAP + blk * 128, 128), 128)])

        j1d[pl.ds(_ACAP - 16, 16)] = jnp.full((16,), dumpj, jnp.int32)
        for q in range(_ACAP // 128):
            qvec = jnp.full((16,), q, jnp.int32)
            for sv in range(8):
                plsc.store_scatter(
                    jrep, [qvec, sv * 16 + iota],
                    j1d[pl.ds(q * 128 + sv * 16, 16)])
        pltpu.sync_copy(jrep, jout_hbm.at[wid])

    return k1


def _build_k2():
    mesh = plsc.VectorSubcoreMesh(core_axis_name="c", subcore_axis_name="s")

    @functools.partial(
        pl.kernel,
        mesh=mesh,
        compiler_params=pltpu.CompilerParams(
            use_tc_tiling_on_sc=True, needs_layout_passes=False),
        out_type=jax.ShapeDtypeStruct((_B // 2, 128), jnp.float32),
        scratch_types=[
            pltpu.VMEM_SHARED(((_B // 2 + 16) * 8,), jnp.int32),
            pltpu.VMEM((_ACAP // 128, 128), jnp.int32),
            pltpu.VMEM((_ACAP // 128, 128), jnp.int32),
            pltpu.VMEM((_ACAP // 128, 128), jnp.int32),
            pltpu.VMEM((_CPW * 128 * 8,), jnp.int32),
            pltpu.VMEM((_CPW * 128,), jnp.int32),
            pltpu.VMEM((2, 128, 128), jnp.float32),
            pltpu.VMEM((64, 128), jnp.float32),
            pltpu.SemaphoreType.DMA,
            pltpu.SemaphoreType.DMA,
        ],
    )
    def k2(jin_hbm, inter_hbm, out_hbm, perm, jrow, jrow8, mval, m8, m1d,
           ring, pk, scsem, gsem):
        cid = lax.axis_index("c")
        sid = lax.axis_index("s")
        wid = sid * 2 + cid
        iota = _iota16()

        for t_local in range(2):
            t = sid * 2 + t_local
            pltpu.sync_copy(jin_hbm.at[t], jrow)

            def fill_body(q, _):
                qv = jnp.full((16,), q, jnp.int32)
                jpw = _CPW * 128
                for sv in range(8):
                    col = sv * 16 + iota
                    jv = plsc.load_gather(jrow, [qv, col])
                    blk = jv // jpw
                    keep = ((blk & 1) == cid) & (jv < _B)
                    slot = ((blk >> 1) * jpw + jv - blk * jpw) * 8
                    slot = jnp.where(keep, slot, (_B // 2 + sid) * 8)
                    plsc.store_scatter(jrow8, [qv, col], slot)
                    plsc.store_scatter(
                        mval, [qv, col], t * _ACAP + q * 128 + col)
                return _

            lax.fori_loop(0, _ACAP // 128, fill_body, 0)
            for q in range(_ACAP // 128):
                pltpu.async_copy(mval.at[q], perm.at[jrow8.at[q]], scsem)
            for q in range(_ACAP // 128):
                pltpu.make_async_copy(
                    mval.at[q], perm.at[jrow8.at[q]], scsem).wait()

        plsc.subcore_barrier()

        j0l = sid * (_CPW * 128)
        pltpu.sync_copy(
            perm.at[pl.ds(pl.multiple_of(j0l * 8, 128), _CPW * 128 * 8)], m8)

        def compact_body(v, _):
            m1d[pl.ds(pl.multiple_of(v * 16, 16), 16)] = plsc.load_gather(
                m8, [(v * 16 + iota) * 8])
            return _

        lax.fori_loop(0, _CPW * 8, compact_body, 0)

        pltpu.async_copy(
            inter_hbm.at[m1d.at[pl.ds(0, 128)]], ring.at[0], gsem)

        def chunk_body(i, carry):
            for pb in range(2):
                r = i * 2 + pb
                pltpu.make_async_copy(
                    inter_hbm.at[m1d.at[pl.ds(pl.multiple_of(r * 128, 128),
                                              128)]],
                    ring.at[pb], gsem).wait()

                @pl.when(r + 1 < _CPW)
                def _():
                    pltpu.async_copy(
                        inter_hbm.at[m1d.at[pl.ds(
                            pl.multiple_of((r + 1) * 128, 128), 128)]],
                        ring.at[1 - pb], gsem)

                pbv = jnp.full((16,), pb, jnp.int32)

                def pack_body(t, _):
                    tv = jnp.full((16,), t, jnp.int32)
                    for sv in range(4):
                        col = sv * 16 + iota
                        a = plsc.load_gather(ring, [pbv, tv * 2, col])
                        b = plsc.load_gather(ring, [pbv, tv * 2 + 1, col])
                        plsc.store_scatter(pk, [tv, col], a)
                        plsc.store_scatter(pk, [tv, 64 + col], b)
                    return _

                lax.fori_loop(0, 64, pack_body, 0)
                pltpu.sync_copy(
                    pk, out_hbm.at[pl.ds(
                        pl.multiple_of((wid * _CPW + r) * 64, 64), 64)])
            return carry

        lax.fori_loop(0, _CPW // 2, chunk_body, 0)

    return k2


def kernel(indices, table):
    idx = indices.reshape(-1).astype(jnp.int32)
    tp = table.T
    tail = jnp.pad(tp[:, _NCOLS - 64:], ((0, 0), (0, 64)))
    inter, jlists = _build_k1()(idx, tp, tail)
    out = _build_k2()(jlists, inter)
    return out.reshape(_BATCH, _N_FIELDS, _DIM)
